# baseline (speedup 1.0000x reference)
import functools

import jax
import jax.numpy as jnp
from jax import lax
from jax.experimental import pallas as pl
from jax.experimental.pallas import tpu as pltpu
from jax.experimental.pallas import tpu_sc as plsc

B = 16384
S = 50
D = 32
L = 16
NC, NS = 2, 16
NW = NC * NS
TPW = B // NW
CT = 2
CI = CT * S
NCHUNK = TPW // CT
NBUF = 4

_mesh = plsc.VectorSubcoreMesh(core_axis_name="c", subcore_axis_name="s")


@functools.partial(
    pl.kernel,
    mesh=_mesh,
    out_type=jax.ShapeDtypeStruct((B, D), jnp.float32),
    scratch_types=[
        pltpu.VMEM((NCHUNK, CI), jnp.int32),
        pltpu.VMEM((NBUF, CI, D), jnp.float32),
        pltpu.VMEM((TPW, D), jnp.float32),
        pltpu.SemaphoreType.DMA((NBUF,)),
    ],
    compiler_params=pltpu.CompilerParams(use_tc_tiling_on_sc=False),
)
def _embed_pool(ids_hbm, table_hbm, out_hbm, idx_v, bufs_v, out_v, sems):
    wid = lax.axis_index("s") * NC + lax.axis_index("c")
    pltpu.sync_copy(ids_hbm.at[pl.ds(wid * NCHUNK, NCHUNK)], idx_v)

    def gather(c, b):
        return pltpu.make_async_copy(
            table_hbm.at[idx_v.at[c]], bufs_v.at[b], sems.at[b])

    for b in range(NBUF):
        gather(b, b).start()

    def group_body(g, carry):
        for b in range(NBUF):
            c = g * NBUF + b
            gather(c, b).wait()
            for t in range(CT):
                acc0 = bufs_v[b, t * S, pl.ds(0, L)]
                acc1 = bufs_v[b, t * S, pl.ds(L, L)]
                for r in range(1, S):
                    acc0 = acc0 + bufs_v[b, t * S + r, pl.ds(0, L)]
                    acc1 = acc1 + bufs_v[b, t * S + r, pl.ds(L, L)]
                out_v[c * CT + t, pl.ds(0, L)] = acc0
                out_v[c * CT + t, pl.ds(L, L)] = acc1
        for b in range(NBUF):
            c = g * NBUF + b

            @pl.when(c + NBUF < NCHUNK)
            def _():
                gather(c + NBUF, b).start()
        return carry

    lax.fori_loop(0, NCHUNK // NBUF, group_body, 0)
    pltpu.sync_copy(out_v, out_hbm.at[pl.ds(wid * TPW, TPW)])


def kernel(token_ids, table):
    ids = token_ids.astype(jnp.int32).reshape(B // CT, CI)
    return _embed_pool(ids, table)


# speedup vs baseline: 1.0233x; 1.0233x over previous
import functools

import jax
import jax.numpy as jnp
from jax import lax
from jax.experimental import pallas as pl
from jax.experimental.pallas import tpu as pltpu
from jax.experimental.pallas import tpu_sc as plsc

B = 16384
S = 50
D = 32
L = 16
NC, NS = 2, 16
NW = NC * NS
TPW = B // NW
CT = 2
CI = CT * S
NCHUNK = TPW // CT
NBUF = 4

_mesh = plsc.VectorSubcoreMesh(core_axis_name="c", subcore_axis_name="s")


@functools.partial(
    pl.kernel,
    mesh=_mesh,
    out_type=jax.ShapeDtypeStruct((B, D), jnp.float32),
    scratch_types=[
        pltpu.VMEM((NCHUNK, CI), jnp.int32),
        pltpu.VMEM((NBUF, CI, D), jnp.float32),
        pltpu.VMEM((TPW, D), jnp.float32),
        pltpu.SemaphoreType.DMA((NBUF,)),
    ],
    compiler_params=pltpu.CompilerParams(use_tc_tiling_on_sc=False),
)
def _embed_pool(ids_hbm, table_hbm, out_hbm, idx_v, bufs_v, out_v, sems):
    wid = lax.axis_index("s") * NC + lax.axis_index("c")
    pltpu.sync_copy(ids_hbm.at[pl.ds(wid * NCHUNK, NCHUNK)], idx_v)

    def gather(c, b):
        return pltpu.make_async_copy(
            table_hbm.at[idx_v.at[c]], bufs_v.at[b], sems.at[b])

    for b in range(NBUF):
        gather(b, b).start()

    def group_body(g, carry):
        for b in range(NBUF):
            c = g * NBUF + b
            gather(c, b).wait()
            for t in range(CT):
                acc0 = bufs_v[b, t * S, pl.ds(0, L)]
                acc1 = bufs_v[b, t * S, pl.ds(L, L)]
                for r in range(1, S):
                    acc0 = acc0 + bufs_v[b, t * S + r, pl.ds(0, L)]
                    acc1 = acc1 + bufs_v[b, t * S + r, pl.ds(L, L)]
                out_v[c * CT + t, pl.ds(0, L)] = acc0
                out_v[c * CT + t, pl.ds(L, L)] = acc1
            q = c - 1 + NBUF

            @pl.when(jnp.logical_and(c >= 1, q < NCHUNK))
            def _():
                gather(q, (b - 1) % NBUF).start()
        return carry

    lax.fori_loop(0, NCHUNK // NBUF, group_body, 0)
    pltpu.sync_copy(out_v, out_hbm.at[pl.ds(wid * TPW, TPW)])


def kernel(token_ids, table):
    ids = token_ids.astype(jnp.int32).reshape(B // CT, CI)
    return _embed_pool(ids, table)


# speedup vs baseline: 1.0575x; 1.0334x over previous
import functools

import jax
import jax.numpy as jnp
from jax import lax
from jax.experimental import pallas as pl
from jax.experimental.pallas import tpu as pltpu
from jax.experimental.pallas import tpu_sc as plsc

B = 16384
S = 50
D = 32
L = 16
NC, NS = 2, 16
NW = NC * NS
TPW = B // NW
CT = 2
CI = CT * S
NCHUNK = TPW // CT
NBUF = 4

_mesh = plsc.VectorSubcoreMesh(core_axis_name="c", subcore_axis_name="s")


@functools.partial(
    pl.kernel,
    mesh=_mesh,
    out_type=jax.ShapeDtypeStruct((B, D), jnp.float32),
    scratch_types=[
        pltpu.VMEM((NCHUNK, CI), jnp.int32),
        pltpu.VMEM((NBUF, CI, D), jnp.float32),
        pltpu.VMEM((TPW, D), jnp.float32),
        pltpu.SemaphoreType.DMA((NBUF,)),
    ],
    compiler_params=pltpu.CompilerParams(use_tc_tiling_on_sc=False),
)
def _embed_pool(ids_hbm, table_hbm, out_hbm, idx_v, bufs_v, out_v, sems):
    wid = lax.axis_index("s") * NC + lax.axis_index("c")
    pltpu.sync_copy(ids_hbm.at[pl.ds(wid * NCHUNK, NCHUNK)], idx_v)

    def gather(c, b):
        return pltpu.make_async_copy(
            table_hbm.at[idx_v.at[c]], bufs_v.at[b], sems.at[b])

    for b in range(NBUF):
        gather(b, b).start()

    def group_body(g, carry):
        for b in range(NBUF):
            c = g * NBUF + b
            gather(c, b).wait()
            for t in range(CT):
                acc0 = bufs_v[b, t * S, pl.ds(0, L)]
                acc1 = bufs_v[b, t * S, pl.ds(L, L)]
                for r in range(1, S):
                    acc0 = acc0 + bufs_v[b, t * S + r, pl.ds(0, L)]
                    acc1 = acc1 + bufs_v[b, t * S + r, pl.ds(L, L)]
                out_v[c * CT + t, pl.ds(0, L)] = acc0
                out_v[c * CT + t, pl.ds(L, L)] = acc1

            @pl.when(c + NBUF < NCHUNK)
            def _():
                gather(c + NBUF, b).start()
        return carry

    lax.fori_loop(0, NCHUNK // NBUF, group_body, 0)
    pltpu.sync_copy(out_v, out_hbm.at[pl.ds(wid * TPW, TPW)])


def kernel(token_ids, table):
    ids = token_ids.astype(jnp.int32).reshape(B // CT, CI)
    return _embed_pool(ids, table)
